# initial kernel scaffold (unmeasured)
import functools

import jax
import jax.numpy as jnp
from jax import lax
from jax.experimental import pallas as pl
from jax.experimental.pallas import tpu as pltpu

P = 4
AXIS = "i"
ADDW = 1024
OUTW = 1024


def _body(s_ref, p_ref, o_ref, commR, commL, own, outb,
          sendR, sendL, recvR, recvL, ldsem, outsem, creditR, creditL,
          *, MB, H):
    d = lax.axis_index(AXIS)
    right = lax.rem(d + 1, P)
    left = lax.rem(d + 3, P)
    s = s_ref[0]

    barrier = pltpu.get_barrier_semaphore()
    for nbr in (left, right):
        pl.semaphore_signal(barrier, inc=1, device_id=(nbr,),
                            device_id_type=pl.DeviceIdType.MESH)
    pl.semaphore_wait(barrier, 2)

    def _load(chunk, col0, dst):
        cp = pltpu.make_async_copy(p_ref.at[chunk, :, pl.ds(col0, H)], dst, ldsem)
        cp.start()
        cp.wait()

    def _accum(comm, slot, chunk, col0):
        _load(chunk, col0, own)
        for j in range(H // ADDW):
            sl = pl.ds(j * ADDW, ADDW)
            comm[slot, :, sl] = comm[slot, :, sl] + own[:, sl]

    def _epilogue(comm, slot, chunk, col0):
        for j in range(H // OUTW):
            sl = pl.ds(j * OUTW, OUTW)
            y = comm[slot, :, sl].astype(jnp.float32) * s
            outb[:, :] = y * (1.0 / (1.0 + jnp.exp(-y)))
            cp = pltpu.make_async_copy(
                outb, o_ref.at[chunk, :, pl.ds(col0 + j * OUTW, OUTW)], outsem)
            cp.start()
            cp.wait()

    _load(lax.rem(d + 3, P), 0, commR.at[0])
    _load(lax.rem(d + 1, P), H, commL.at[0])

    for t in range(2 * P - 2):
        a, b = t % 2, (t + 1) % 2
        if t > 0:
            pl.semaphore_wait(creditR, 1)
            pl.semaphore_wait(creditL, 1)
        rR = pltpu.make_async_remote_copy(
            src_ref=commR.at[a], dst_ref=commR.at[b],
            send_sem=sendR.at[a], recv_sem=recvR.at[t],
            device_id=(right,), device_id_type=pl.DeviceIdType.MESH)
        rL = pltpu.make_async_remote_copy(
            src_ref=commL.at[a], dst_ref=commL.at[b],
            send_sem=sendL.at[a], recv_sem=recvL.at[t],
            device_id=(left,), device_id_type=pl.DeviceIdType.MESH)
        rR.start()
        rL.start()
        if t == P - 1:
            _epilogue(commR, a, d, 0)
            _epilogue(commL, a, d, H)
        rR.wait()
        rL.wait()
        if t < P - 1:
            _accum(commR, b, lax.rem(d + (P - 2 - t) % P, P), 0)
            _accum(commL, b, lax.rem(d + 2 + t, P), H)
        else:
            g = t - (P - 1)
            _epilogue(commR, b, lax.rem(d + P - 1 - g, P), 0)
            _epilogue(commL, b, lax.rem(d + 1 + g, P), H)
        if t < 2 * P - 3:
            pl.semaphore_signal(creditR, inc=1, device_id=(left,),
                                device_id_type=pl.DeviceIdType.MESH)
            pl.semaphore_signal(creditL, inc=1, device_id=(right,),
                                device_id_type=pl.DeviceIdType.MESH)


def kernel(x, w_mat, scale_x, scale_w):
    M = x.shape[0]
    N = w_mat.shape[1]
    MB = M // P
    H = N // 2

    p16 = jnp.dot(x.astype(jnp.bfloat16), w_mat.astype(jnp.bfloat16),
                  preferred_element_type=jnp.bfloat16)
    p16 = p16.reshape(P, MB, N)
    s_arr = (scale_x * scale_w).astype(jnp.float32).reshape(1)

    body = functools.partial(_body, MB=MB, H=H)
    out = pl.pallas_call(
        body,
        out_shape=jax.ShapeDtypeStruct((P, MB, N), jnp.float32),
        in_specs=[
            pl.BlockSpec(memory_space=pltpu.SMEM),
            pl.BlockSpec(memory_space=pltpu.ANY),
        ],
        out_specs=pl.BlockSpec(memory_space=pltpu.ANY),
        scratch_shapes=[
            pltpu.VMEM((2, MB, H), jnp.bfloat16),
            pltpu.VMEM((2, MB, H), jnp.bfloat16),
            pltpu.VMEM((MB, H), jnp.bfloat16),
            pltpu.VMEM((MB, OUTW), jnp.float32),
            pltpu.SemaphoreType.DMA((2,)),
            pltpu.SemaphoreType.DMA((2,)),
            pltpu.SemaphoreType.DMA((2 * P - 2,)),
            pltpu.SemaphoreType.DMA((2 * P - 2,)),
            pltpu.SemaphoreType.DMA,
            pltpu.SemaphoreType.DMA,
            pltpu.SemaphoreType.REGULAR,
            pltpu.SemaphoreType.REGULAR,
        ],
        compiler_params=pltpu.CompilerParams(collective_id=0),
    )(s_arr, p16)
    return out.reshape(M, N)


# baseline (device time: 848611 ns/iter reference)
import functools

import jax
import jax.numpy as jnp
from jax import lax
from jax.experimental import pallas as pl
from jax.experimental.pallas import tpu as pltpu

P = 4
AXIS = "i"
ADDW = 1024
OUTW = 1024


def _body(s_ref, p_ref, o_ref, commR, commL, own, outb,
          sendR, sendL, recvR, recvL, ldsem, outsem, creditR, creditL,
          *, MB, H):
    d = lax.axis_index(AXIS)
    right = lax.rem(d + 1, P)
    left = lax.rem(d + 3, P)
    s = s_ref[0]

    barrier = pltpu.get_barrier_semaphore()
    for nbr in (left, right):
        pl.semaphore_signal(barrier, inc=1, device_id=(nbr,),
                            device_id_type=pl.DeviceIdType.MESH)
    pl.semaphore_wait(barrier, 2)

    def _load(chunk, col0, dst):
        cp = pltpu.make_async_copy(p_ref.at[chunk, :, pl.ds(col0, H)], dst, ldsem)
        cp.start()
        cp.wait()

    def _accum(comm, slot, chunk, col0):
        _load(chunk, col0, own)
        for j in range(H // ADDW):
            sl = pl.ds(j * ADDW, ADDW)
            comm[slot, :, sl] = comm[slot, :, sl] + own[:, sl]

    def _epilogue(comm, slot, chunk, col0):
        for j in range(H // OUTW):
            sl = pl.ds(j * OUTW, OUTW)
            y = comm[slot, :, sl].astype(jnp.float32) * s
            outb[:, :] = y * (1.0 / (1.0 + jnp.exp(-y)))
            cp = pltpu.make_async_copy(
                outb, o_ref.at[chunk, :, pl.ds(col0 + j * OUTW, OUTW)], outsem)
            cp.start()
            cp.wait()

    _load(lax.rem(d + 3, P), 0, commR.at[0])
    _load(lax.rem(d + 1, P), H, commL.at[0])

    for t in range(2 * P - 2):
        a, b = t % 2, (t + 1) % 2
        if t > 0:
            pl.semaphore_wait(creditR, 1)
            pl.semaphore_wait(creditL, 1)
        rR = pltpu.make_async_remote_copy(
            src_ref=commR.at[a], dst_ref=commR.at[b],
            send_sem=sendR.at[a], recv_sem=recvR.at[t],
            device_id=(right,), device_id_type=pl.DeviceIdType.MESH)
        rL = pltpu.make_async_remote_copy(
            src_ref=commL.at[a], dst_ref=commL.at[b],
            send_sem=sendL.at[a], recv_sem=recvL.at[t],
            device_id=(left,), device_id_type=pl.DeviceIdType.MESH)
        rR.start()
        rL.start()
        if t == P - 1:
            _epilogue(commR, a, d, 0)
            _epilogue(commL, a, d, H)
        rR.wait()
        rL.wait()
        if t < P - 1:
            _accum(commR, b, lax.rem(d + (P - 2 - t) % P, P), 0)
            _accum(commL, b, lax.rem(d + 2 + t, P), H)
        else:
            g = t - (P - 1)
            _epilogue(commR, b, lax.rem(d + P - 1 - g, P), 0)
            _epilogue(commL, b, lax.rem(d + 1 + g, P), H)
        if t < 2 * P - 3:
            pl.semaphore_signal(creditR, inc=1, device_id=(left,),
                                device_id_type=pl.DeviceIdType.MESH)
            pl.semaphore_signal(creditL, inc=1, device_id=(right,),
                                device_id_type=pl.DeviceIdType.MESH)


def kernel(x, w_mat, scale_x, scale_w):
    M = x.shape[0]
    N = w_mat.shape[1]
    MB = M // P
    H = N // 2

    p16 = jnp.dot(x.astype(jnp.bfloat16), w_mat.astype(jnp.bfloat16),
                  preferred_element_type=jnp.bfloat16)
    p16 = p16.reshape(P, MB, N)
    s_arr = (scale_x * scale_w).astype(jnp.float32).reshape(1)

    body = functools.partial(_body, MB=MB, H=H)
    out = pl.pallas_call(
        body,
        out_shape=jax.ShapeDtypeStruct((P, MB, N), jnp.float32),
        in_specs=[
            pl.BlockSpec(memory_space=pltpu.SMEM),
            pl.BlockSpec(memory_space=pl.ANY),
        ],
        out_specs=pl.BlockSpec(memory_space=pl.ANY),
        scratch_shapes=[
            pltpu.VMEM((2, MB, H), jnp.bfloat16),
            pltpu.VMEM((2, MB, H), jnp.bfloat16),
            pltpu.VMEM((MB, H), jnp.bfloat16),
            pltpu.VMEM((MB, OUTW), jnp.float32),
            pltpu.SemaphoreType.DMA((2,)),
            pltpu.SemaphoreType.DMA((2,)),
            pltpu.SemaphoreType.DMA((2 * P - 2,)),
            pltpu.SemaphoreType.DMA((2 * P - 2,)),
            pltpu.SemaphoreType.DMA,
            pltpu.SemaphoreType.DMA,
            pltpu.SemaphoreType.REGULAR,
            pltpu.SemaphoreType.REGULAR,
        ],
        compiler_params=pltpu.CompilerParams(
            collective_id=0, vmem_limit_bytes=60 * 1024 * 1024),
    )(s_arr, p16)
    return out.reshape(M, N)


# device time: 778061 ns/iter; 1.0907x vs baseline; 1.0907x over previous
import functools

import jax
import jax.numpy as jnp
from jax import lax
from jax.experimental import pallas as pl
from jax.experimental.pallas import tpu as pltpu

P = 4
AXIS = "i"
ADDW = 1024
OUTW = 1024


def _body(s_ref, p_ref, o_ref, commR, commL, ownR, ownL, outb,
          sendR, sendL, recvR, recvL, ldsemR, ldsemL, outsem,
          creditR, creditL, *, MB, H):
    d = lax.axis_index(AXIS)
    right = lax.rem(d + 1, P)
    left = lax.rem(d + 3, P)
    s = s_ref[0]

    def _load(chunk, col0, dst, sem):
        cp = pltpu.make_async_copy(p_ref.at[chunk, :, pl.ds(col0, H)], dst, sem)
        cp.start()
        return cp

    def _epilogue(comm, slot, chunk, col0):
        for j in range(H // OUTW):
            sl = pl.ds(j * OUTW, OUTW)
            y = comm[slot, :, sl].astype(jnp.float32) * s
            outb[:, :] = y * (1.0 / (1.0 + jnp.exp(-y)))
            cp = pltpu.make_async_copy(
                outb, o_ref.at[chunk, :, pl.ds(col0 + j * OUTW, OUTW)], outsem)
            cp.start()
            cp.wait()

    ld0 = _load(lax.rem(d + 3, P), 0, commR.at[0], ldsemR)
    ld1 = _load(lax.rem(d + 1, P), H, commL.at[0], ldsemL)
    ld0.wait()
    ld1.wait()

    barrier = pltpu.get_barrier_semaphore()
    for nbr in (left, right):
        pl.semaphore_signal(barrier, inc=1, device_id=(nbr,),
                            device_id_type=pl.DeviceIdType.MESH)
    pl.semaphore_wait(barrier, 2)

    for t in range(2 * P - 2):
        a, b = t % 2, (t + 1) % 2
        if t > 0:
            pl.semaphore_wait(creditR, 1)
            pl.semaphore_wait(creditL, 1)
        rR = pltpu.make_async_remote_copy(
            src_ref=commR.at[a], dst_ref=commR.at[b],
            send_sem=sendR.at[a], recv_sem=recvR.at[t],
            device_id=(right,), device_id_type=pl.DeviceIdType.MESH)
        rL = pltpu.make_async_remote_copy(
            src_ref=commL.at[a], dst_ref=commL.at[b],
            send_sem=sendL.at[a], recv_sem=recvL.at[t],
            device_id=(left,), device_id_type=pl.DeviceIdType.MESH)
        rR.start()
        rL.start()
        if t < P - 1:
            ldR = _load(lax.rem(d + (P - 2 - t) % P, P), 0, ownR, ldsemR)
            ldL = _load(lax.rem(d + 2 + t, P), H, ownL, ldsemL)
        elif t == P - 1:
            _epilogue(commR, a, d, 0)
            _epilogue(commL, a, d, H)
        else:
            g = t - P
            _epilogue(commR, a, lax.rem(d + P - 1 - g, P), 0)
            _epilogue(commL, a, lax.rem(d + 1 + g, P), H)
        rR.wait()
        rL.wait()
        if t < P - 1:
            ldR.wait()
            ldL.wait()
            for j in range(H // ADDW):
                sl = pl.ds(j * ADDW, ADDW)
                commR[b, :, sl] = commR[b, :, sl] + ownR[:, sl]
            for j in range(H // ADDW):
                sl = pl.ds(j * ADDW, ADDW)
                commL[b, :, sl] = commL[b, :, sl] + ownL[:, sl]
        if t < 2 * P - 3:
            pl.semaphore_signal(creditR, inc=1, device_id=(left,),
                                device_id_type=pl.DeviceIdType.MESH)
            pl.semaphore_signal(creditL, inc=1, device_id=(right,),
                                device_id_type=pl.DeviceIdType.MESH)

    _epilogue(commR, 0, lax.rem(d + 1, P), 0)
    _epilogue(commL, 0, lax.rem(d + 3, P), H)


def kernel(x, w_mat, scale_x, scale_w):
    M = x.shape[0]
    N = w_mat.shape[1]
    MB = M // P
    H = N // 2

    p16 = jnp.dot(x.astype(jnp.bfloat16), w_mat.astype(jnp.bfloat16),
                  preferred_element_type=jnp.bfloat16)
    p16 = p16.reshape(P, MB, N)
    s_arr = (scale_x * scale_w).astype(jnp.float32).reshape(1)

    body = functools.partial(_body, MB=MB, H=H)
    out = pl.pallas_call(
        body,
        out_shape=jax.ShapeDtypeStruct((P, MB, N), jnp.float32),
        in_specs=[
            pl.BlockSpec(memory_space=pltpu.SMEM),
            pl.BlockSpec(memory_space=pl.ANY),
        ],
        out_specs=pl.BlockSpec(memory_space=pl.ANY),
        scratch_shapes=[
            pltpu.VMEM((2, MB, H), jnp.bfloat16),
            pltpu.VMEM((2, MB, H), jnp.bfloat16),
            pltpu.VMEM((MB, H), jnp.bfloat16),
            pltpu.VMEM((MB, H), jnp.bfloat16),
            pltpu.VMEM((MB, OUTW), jnp.float32),
            pltpu.SemaphoreType.DMA((2,)),
            pltpu.SemaphoreType.DMA((2,)),
            pltpu.SemaphoreType.DMA((2 * P - 2,)),
            pltpu.SemaphoreType.DMA((2 * P - 2,)),
            pltpu.SemaphoreType.DMA,
            pltpu.SemaphoreType.DMA,
            pltpu.SemaphoreType.DMA,
            pltpu.SemaphoreType.REGULAR,
            pltpu.SemaphoreType.REGULAR,
        ],
        compiler_params=pltpu.CompilerParams(
            collective_id=0, vmem_limit_bytes=60 * 1024 * 1024),
    )(s_arr, p16)
    return out.reshape(M, N)


# device time: 776901 ns/iter; 1.0923x vs baseline; 1.0015x over previous
import functools

import jax
import jax.numpy as jnp
from jax import lax
from jax.experimental import pallas as pl
from jax.experimental.pallas import tpu as pltpu

P = 4
AXIS = "i"
ADDW = 2048
OUTW = 1024


def _body(s_ref, p_ref, o_ref, commR, commL, ownR, ownL, outb,
          sendR, sendL, recvR, recvL, ldsemR, ldsemL, outsem,
          creditR, creditL, *, MB, H):
    d = lax.axis_index(AXIS)
    right = lax.rem(d + 1, P)
    left = lax.rem(d + 3, P)
    s = s_ref[0]

    def _load(chunk, col0, dst, sem):
        cp = pltpu.make_async_copy(p_ref.at[chunk, :, pl.ds(col0, H)], dst, sem)
        cp.start()
        return cp

    def _epilogue(comm, slot, chunk, col0):
        for j in range(H // OUTW):
            sl = pl.ds(j * OUTW, OUTW)
            y = comm[slot, :, sl].astype(jnp.float32) * s
            outb[:, :] = y * (1.0 / (1.0 + jnp.exp(-y)))
            cp = pltpu.make_async_copy(
                outb, o_ref.at[chunk, :, pl.ds(col0 + j * OUTW, OUTW)], outsem)
            cp.start()
            cp.wait()

    ld0 = _load(lax.rem(d + 3, P), 0, commR.at[0], ldsemR)
    ld1 = _load(lax.rem(d + 1, P), H, commL.at[0], ldsemL)
    ld0.wait()
    ld1.wait()

    barrier = pltpu.get_barrier_semaphore()
    for nbr in (left, right):
        pl.semaphore_signal(barrier, inc=1, device_id=(nbr,),
                            device_id_type=pl.DeviceIdType.MESH)
    pl.semaphore_wait(barrier, 2)

    for t in range(2 * P - 2):
        a, b = t % 2, (t + 1) % 2
        if t > 0:
            pl.semaphore_wait(creditR, 1)
            pl.semaphore_wait(creditL, 1)
        rR = pltpu.make_async_remote_copy(
            src_ref=commR.at[a], dst_ref=commR.at[b],
            send_sem=sendR.at[a], recv_sem=recvR.at[t],
            device_id=(right,), device_id_type=pl.DeviceIdType.MESH)
        rL = pltpu.make_async_remote_copy(
            src_ref=commL.at[a], dst_ref=commL.at[b],
            send_sem=sendL.at[a], recv_sem=recvL.at[t],
            device_id=(left,), device_id_type=pl.DeviceIdType.MESH)
        rR.start()
        rL.start()
        if t < P - 1:
            ldR = _load(lax.rem(d + (P - 2 - t) % P, P), 0, ownR, ldsemR)
            ldL = _load(lax.rem(d + 2 + t, P), H, ownL, ldsemL)
        elif t == P - 1:
            _epilogue(commR, a, d, 0)
            _epilogue(commL, a, d, H)
        else:
            g = t - P
            _epilogue(commR, a, lax.rem(d + P - 1 - g, P), 0)
            _epilogue(commL, a, lax.rem(d + 1 + g, P), H)
        rR.wait()
        rL.wait()
        if t < 2 * P - 3:
            pl.semaphore_signal(creditR, inc=1, device_id=(left,),
                                device_id_type=pl.DeviceIdType.MESH)
            pl.semaphore_signal(creditL, inc=1, device_id=(right,),
                                device_id_type=pl.DeviceIdType.MESH)
        if t < P - 1:
            ldR.wait()
            ldL.wait()
            for j in range(H // ADDW):
                sl = pl.ds(j * ADDW, ADDW)
                commR[b, :, sl] = commR[b, :, sl] + ownR[:, sl]
            for j in range(H // ADDW):
                sl = pl.ds(j * ADDW, ADDW)
                commL[b, :, sl] = commL[b, :, sl] + ownL[:, sl]

    _epilogue(commR, 0, lax.rem(d + 1, P), 0)
    _epilogue(commL, 0, lax.rem(d + 3, P), H)


def kernel(x, w_mat, scale_x, scale_w):
    M = x.shape[0]
    N = w_mat.shape[1]
    MB = M // P
    H = N // 2

    p16 = jnp.dot(x.astype(jnp.bfloat16), w_mat.astype(jnp.bfloat16),
                  preferred_element_type=jnp.bfloat16)
    p16 = p16.reshape(P, MB, N)
    s_arr = (scale_x * scale_w).astype(jnp.float32).reshape(1)

    body = functools.partial(_body, MB=MB, H=H)
    out = pl.pallas_call(
        body,
        out_shape=jax.ShapeDtypeStruct((P, MB, N), jnp.float32),
        in_specs=[
            pl.BlockSpec(memory_space=pltpu.SMEM),
            pl.BlockSpec(memory_space=pl.ANY),
        ],
        out_specs=pl.BlockSpec(memory_space=pl.ANY),
        scratch_shapes=[
            pltpu.VMEM((2, MB, H), jnp.bfloat16),
            pltpu.VMEM((2, MB, H), jnp.bfloat16),
            pltpu.VMEM((MB, H), jnp.bfloat16),
            pltpu.VMEM((MB, H), jnp.bfloat16),
            pltpu.VMEM((MB, OUTW), jnp.float32),
            pltpu.SemaphoreType.DMA((2,)),
            pltpu.SemaphoreType.DMA((2,)),
            pltpu.SemaphoreType.DMA((2 * P - 2,)),
            pltpu.SemaphoreType.DMA((2 * P - 2,)),
            pltpu.SemaphoreType.DMA,
            pltpu.SemaphoreType.DMA,
            pltpu.SemaphoreType.DMA,
            pltpu.SemaphoreType.REGULAR,
            pltpu.SemaphoreType.REGULAR,
        ],
        compiler_params=pltpu.CompilerParams(
            collective_id=0, vmem_limit_bytes=60 * 1024 * 1024),
    )(s_arr, p16)
    return out.reshape(M, N)


# device time: 732536 ns/iter; 1.1585x vs baseline; 1.0606x over previous
import functools

import jax
import jax.numpy as jnp
from jax import lax
from jax.experimental import pallas as pl
from jax.experimental.pallas import tpu as pltpu

P = 4
AXIS = "i"
ADDW = 2048
OUTW = 1024
_USE_EXPLICIT_BARRIER = True


def _body(s_ref, p_ref, o_ref, commR, commL, ownR, ownL, outb,
          sendR, sendL, recvR, recvL, ldsemR, ldsemL, outsem,
          creditR, creditL, *, MB, H):
    d = lax.axis_index(AXIS)
    right = lax.rem(d + 1, P)
    left = lax.rem(d + 3, P)
    s = s_ref[0]

    def _load(chunk, col0, dst, sem):
        cp = pltpu.make_async_copy(
            p_ref.at[pl.ds(chunk * MB, MB), pl.ds(col0, H)], dst, sem)
        cp.start()
        return cp

    def _epilogue(comm, slot, chunk, col0, off=0, width=None):
        width = H if width is None else width
        for j in range(width // OUTW):
            sl = pl.ds(off + j * OUTW, OUTW)
            y = comm[slot, :, sl].astype(jnp.float32) * s
            outb[:, :] = y * (1.0 / (1.0 + jnp.exp(-y)))
            cp = pltpu.make_async_copy(
                outb,
                o_ref.at[pl.ds(chunk * MB, MB),
                         pl.ds(col0 + off + j * OUTW, OUTW)],
                outsem)
            cp.start()
            cp.wait()

    ld0 = _load(lax.rem(d + 3, P), 0, commR.at[0], ldsemR)
    ld1 = _load(lax.rem(d + 1, P), H, commL.at[0], ldsemL)
    ld0.wait()
    ld1.wait()

    if _USE_EXPLICIT_BARRIER:
        barrier = pltpu.get_barrier_semaphore()
        for nbr in (left, right):
            pl.semaphore_signal(barrier, inc=1, device_id=(nbr,),
                                device_id_type=pl.DeviceIdType.MESH)
        pl.semaphore_wait(barrier, 2)

    for t in range(2 * P - 3):
        a, b = t % 2, (t + 1) % 2
        if t > 0:
            pl.semaphore_wait(creditR, 1)
            pl.semaphore_wait(creditL, 1)
        rR = pltpu.make_async_remote_copy(
            src_ref=commR.at[a], dst_ref=commR.at[b],
            send_sem=sendR.at[a], recv_sem=recvR.at[t],
            device_id=(right,), device_id_type=pl.DeviceIdType.MESH)
        rL = pltpu.make_async_remote_copy(
            src_ref=commL.at[a], dst_ref=commL.at[b],
            send_sem=sendL.at[a], recv_sem=recvL.at[t],
            device_id=(left,), device_id_type=pl.DeviceIdType.MESH)
        rR.start()
        rL.start()
        if t < P - 1:
            ldR = _load(lax.rem(d + (P - 2 - t) % P, P), 0, ownR, ldsemR)
            ldL = _load(lax.rem(d + 2 + t, P), H, ownL, ldsemL)
        elif t == P - 1:
            _epilogue(commR, a, d, 0)
            _epilogue(commL, a, d, H)
        else:
            g = t - P
            _epilogue(commR, a, lax.rem(d + P - 1 - g, P), 0)
            _epilogue(commL, a, lax.rem(d + 1 + g, P), H)
        rR.wait()
        rL.wait()
        if t < 2 * P - 3:
            pl.semaphore_signal(creditR, inc=1, device_id=(left,),
                                device_id_type=pl.DeviceIdType.MESH)
            pl.semaphore_signal(creditL, inc=1, device_id=(right,),
                                device_id_type=pl.DeviceIdType.MESH)
        if t < P - 1:
            ldR.wait()
            ldL.wait()
            for j in range(H // ADDW):
                sl = pl.ds(j * ADDW, ADDW)
                commR[b, :, sl] = commR[b, :, sl] + ownR[:, sl]
            for j in range(H // ADDW):
                sl = pl.ds(j * ADDW, ADDW)
                commL[b, :, sl] = commL[b, :, sl] + ownL[:, sl]

    t = 2 * P - 3
    a, b = t % 2, (t + 1) % 2
    H2 = H // 2
    pl.semaphore_wait(creditR, 1)
    pl.semaphore_wait(creditL, 1)
    subs = []
    for k in range(2):
        off = pl.ds(k * H2, H2)
        rRk = pltpu.make_async_remote_copy(
            src_ref=commR.at[a, :, off], dst_ref=commR.at[b, :, off],
            send_sem=sendR.at[a if k == 0 else b], recv_sem=recvR.at[t + k],
            device_id=(right,), device_id_type=pl.DeviceIdType.MESH)
        rLk = pltpu.make_async_remote_copy(
            src_ref=commL.at[a, :, off], dst_ref=commL.at[b, :, off],
            send_sem=sendL.at[a if k == 0 else b], recv_sem=recvL.at[t + k],
            device_id=(left,), device_id_type=pl.DeviceIdType.MESH)
        rRk.start()
        rLk.start()
        subs.append((rRk, rLk))
    g = t - P
    _epilogue(commR, a, lax.rem(d + P - 1 - g, P), 0)
    _epilogue(commL, a, lax.rem(d + 1 + g, P), H)
    cRf = lax.rem(d + 1, P)
    cLf = lax.rem(d + 3, P)
    for k, (rRk, rLk) in enumerate(subs):
        rRk.wait()
        rLk.wait()
        _epilogue(commR, b, cRf, 0, off=k * H2, width=H2)
        _epilogue(commL, b, cLf, H, off=k * H2, width=H2)


def kernel(x, w_mat, scale_x, scale_w):
    M = x.shape[0]
    N = w_mat.shape[1]
    MB = M // P
    H = N // 2

    p16 = jnp.dot(x.astype(jnp.float8_e4m3fn), w_mat.astype(jnp.float8_e4m3fn),
                  preferred_element_type=jnp.bfloat16)
    s_arr = (scale_x * scale_w).astype(jnp.float32).reshape(1)

    body = functools.partial(_body, MB=MB, H=H)
    out = pl.pallas_call(
        body,
        out_shape=jax.ShapeDtypeStruct((M, N), jnp.float32),
        in_specs=[
            pl.BlockSpec(memory_space=pltpu.SMEM),
            pl.BlockSpec(memory_space=pl.ANY),
        ],
        out_specs=pl.BlockSpec(memory_space=pl.ANY),
        scratch_shapes=[
            pltpu.VMEM((2, MB, H), jnp.bfloat16),
            pltpu.VMEM((2, MB, H), jnp.bfloat16),
            pltpu.VMEM((MB, H), jnp.bfloat16),
            pltpu.VMEM((MB, H), jnp.bfloat16),
            pltpu.VMEM((MB, OUTW), jnp.float32),
            pltpu.SemaphoreType.DMA((2,)),
            pltpu.SemaphoreType.DMA((2,)),
            pltpu.SemaphoreType.DMA((2 * P - 1,)),
            pltpu.SemaphoreType.DMA((2 * P - 1,)),
            pltpu.SemaphoreType.DMA,
            pltpu.SemaphoreType.DMA,
            pltpu.SemaphoreType.DMA,
            pltpu.SemaphoreType.REGULAR,
            pltpu.SemaphoreType.REGULAR,
        ],
        compiler_params=pltpu.CompilerParams(
            vmem_limit_bytes=60 * 1024 * 1024,
            **({"collective_id": 0} if _USE_EXPLICIT_BARRIER else {})),
    )(s_arr, p16)
    return out
